# baseline (speedup 1.0000x reference)
import functools

import jax
import jax.numpy as jnp
from jax import lax
from jax.experimental import pallas as pl
from jax.experimental.pallas import tpu as pltpu
from jax.experimental.pallas import tpu_sc as plsc

_ALPHA = 4


_FAST_TB = 8


def _fast_copy_body(src, dst, buf0, buf1, g0, g1, s0, s1):
    C, T = src.shape[0], src.shape[1]
    bufs = (buf0, buf1)
    gsems = (g0, g1)
    ssems = (s0, s1)
    n = C * (T // _FAST_TB)
    gathers = [None, None]
    scatters = [None, None]
    for k in range(n):
        b = k % 2
        c, t = k // (T // _FAST_TB), k % (T // _FAST_TB)
        rows = pl.ds(t * _FAST_TB, _FAST_TB)
        if scatters[b] is not None:
            scatters[b].wait()
        gathers[b] = pltpu.make_async_copy(src.at[c, rows], bufs[b], gsems[b])
        gathers[b].start()
        gathers[b].wait()
        scatters[b] = pltpu.make_async_copy(bufs[b], dst.at[c, rows], ssems[b])
        scatters[b].start()
    for b in range(2):
        if scatters[b] is not None:
            scatters[b].wait()


def _fast_copy(frames):
    C, T, H, W = frames.shape
    return pl.pallas_call(
        _fast_copy_body,
        out_shape=jax.ShapeDtypeStruct(frames.shape, frames.dtype),
        in_specs=[pl.BlockSpec(memory_space=pl.ANY)],
        out_specs=pl.BlockSpec(memory_space=pl.ANY),
        scratch_shapes=[
            pltpu.VMEM((_FAST_TB, H, W), jnp.float32),
            pltpu.VMEM((_FAST_TB, H, W), jnp.float32),
            pltpu.SemaphoreType.DMA,
            pltpu.SemaphoreType.DMA,
            pltpu.SemaphoreType.DMA,
            pltpu.SemaphoreType.DMA,
        ],
    )(frames)


def kernel(frames):
    C, T, H, W = frames.shape
    TS = T // _ALPHA
    NSLICES = C * TS
    CHUNKS = 4
    RB = H // CHUNKS

    info = plsc.get_sparse_core_info()
    NC, NS = info.num_cores, info.num_subcores
    NW = NC * NS
    NTASK = NSLICES * CHUNKS
    TPW = NTASK // NW

    mesh = plsc.VectorSubcoreMesh(core_axis_name="c", subcore_axis_name="s")

    @functools.partial(
        pl.kernel,
        mesh=mesh,
        out_type=jax.ShapeDtypeStruct((C, TS, H, W), jnp.float32),
        scratch_types=[
            pltpu.VMEM((RB, W), jnp.float32),
            pltpu.VMEM((RB, W), jnp.float32),
            pltpu.SemaphoreType.DMA,
            pltpu.SemaphoreType.DMA,
            pltpu.SemaphoreType.DMA,
            pltpu.SemaphoreType.DMA,
        ],
        compiler_params=pltpu.CompilerParams(use_tc_tiling_on_sc=True),
    )
    def gather_slices(src_hbm, out_hbm, buf0, buf1, g0, g1, s0, s1):
        wid = lax.axis_index("s") * NC + lax.axis_index("c")
        bufs = (buf0, buf1)
        gsems = (g0, g1)
        ssems = (s0, s1)

        def task_refs(k):
            task = wid * TPW + k
            sl = task // CHUNKS
            q = task % CHUNKS
            c = sl // TS
            t = sl % TS
            t_src = (t * (T - 1)) // (TS - 1)
            rows = pl.ds(q * RB, RB)
            return (src_hbm.at[c, t_src, rows, :],
                    out_hbm.at[c, t, rows, :])

        gathers = [None, None]
        scatters = [None, None]
        for k in range(TPW):
            b = k % 2
            src_ref, dst_ref = task_refs(k)
            if scatters[b] is not None:
                scatters[b].wait()
            gathers[b] = pltpu.async_copy(src_ref, bufs[b], gsems[b])
            gathers[b].wait()
            scatters[b] = pltpu.async_copy(bufs[b], dst_ref, ssems[b])
        for b in range(2):
            if scatters[b] is not None:
                scatters[b].wait()

    slow = gather_slices(frames)
    fast = _fast_copy(frames)
    return (slow, fast)


# speedup vs baseline: 1.3216x; 1.3216x over previous
import functools

import jax
import jax.numpy as jnp
from jax import lax
from jax.experimental import pallas as pl
from jax.experimental.pallas import tpu as pltpu
from jax.experimental.pallas import tpu_sc as plsc

_ALPHA = 4


_FAST_TB = 8


def _fast_copy_body(src, dst, *rest):
    C, T = src.shape[0], src.shape[1]
    npc = T // _FAST_TB
    n = C * npc
    bufs, gsem, ssem = rest[:n], rest[n], rest[n + 1]
    gathers = []
    for k in range(n):
        c, t = k // npc, k % npc
        rows = pl.ds(t * _FAST_TB, _FAST_TB)
        cp = pltpu.make_async_copy(src.at[c, rows], bufs[k], gsem.at[k])
        cp.start()
        gathers.append((cp, c, rows))
    scatters = []
    for k, (cp, c, rows) in enumerate(gathers):
        cp.wait()
        s = pltpu.make_async_copy(bufs[k], dst.at[c, rows], ssem.at[k])
        s.start()
        scatters.append(s)
    for s in scatters:
        s.wait()


def _fast_copy(frames):
    C, T, H, W = frames.shape
    n = C * (T // _FAST_TB)
    return pl.pallas_call(
        _fast_copy_body,
        out_shape=jax.ShapeDtypeStruct(frames.shape, frames.dtype),
        in_specs=[pl.BlockSpec(memory_space=pl.ANY)],
        out_specs=pl.BlockSpec(memory_space=pl.ANY),
        scratch_shapes=(
            [pltpu.VMEM((_FAST_TB, H, W), jnp.float32) for _ in range(n)]
            + [pltpu.SemaphoreType.DMA((n,)), pltpu.SemaphoreType.DMA((n,))]
        ),
    )(frames)


def kernel(frames):
    C, T, H, W = frames.shape
    TS = T // _ALPHA
    NSLICES = C * TS
    CHUNKS = 4
    RB = H // CHUNKS

    info = plsc.get_sparse_core_info()
    NC, NS = info.num_cores, info.num_subcores
    NW = NC * NS
    NTASK = NSLICES * CHUNKS
    TPW = NTASK // NW

    mesh = plsc.VectorSubcoreMesh(core_axis_name="c", subcore_axis_name="s")

    @functools.partial(
        pl.kernel,
        mesh=mesh,
        out_type=jax.ShapeDtypeStruct((C, TS, H, W), jnp.float32),
        scratch_types=[
            pltpu.VMEM((RB, W), jnp.float32),
            pltpu.VMEM((RB, W), jnp.float32),
            pltpu.SemaphoreType.DMA,
            pltpu.SemaphoreType.DMA,
            pltpu.SemaphoreType.DMA,
            pltpu.SemaphoreType.DMA,
        ],
        compiler_params=pltpu.CompilerParams(use_tc_tiling_on_sc=True),
    )
    def gather_slices(src_hbm, out_hbm, buf0, buf1, g0, g1, s0, s1):
        wid = lax.axis_index("s") * NC + lax.axis_index("c")
        bufs = (buf0, buf1)
        gsems = (g0, g1)
        ssems = (s0, s1)

        def task_refs(k):
            task = wid * TPW + k
            sl = task // CHUNKS
            q = task % CHUNKS
            c = sl // TS
            t = sl % TS
            t_src = (t * (T - 1)) // (TS - 1)
            rows = pl.ds(q * RB, RB)
            return (src_hbm.at[c, t_src, rows, :],
                    out_hbm.at[c, t, rows, :])

        gathers = [None, None]
        scatters = [None, None]
        for k in range(TPW):
            b = k % 2
            src_ref, dst_ref = task_refs(k)
            if scatters[b] is not None:
                scatters[b].wait()
            gathers[b] = pltpu.async_copy(src_ref, bufs[b], gsems[b])
            gathers[b].wait()
            scatters[b] = pltpu.async_copy(bufs[b], dst_ref, ssems[b])
        for b in range(2):
            if scatters[b] is not None:
                scatters[b].wait()

    slow = gather_slices(frames)
    fast = _fast_copy(frames)
    return (slow, fast)
